# baseline (speedup 1.0000x reference)
import functools

import jax
import jax.numpy as jnp
from jax import lax
from jax.experimental import pallas as pl
from jax.experimental.pallas import tpu as pltpu
from jax.experimental.pallas import tpu_sc as plsc

_N_EDGES = 209982
_N_NODES = 10000
_N_PAD = 10240
_NC = 2
_NS = 16
_NW = _NC * _NS
_SENT = _N_PAD - 1

_NROWS = _N_EDGES // 128
_RPT = 104
_RLAST = _NROWS - _RPT * (_NS - 1)
_TAIL_OFF = _NROWS * 128
_TAIL = _N_EDGES - _TAIL_OFF

_E_PW = 6560
_NODES_PT = _N_PAD // _NS

_DST_SKEW = _N_EDGES % 8
_DST_BASE = _N_EDGES - _DST_SKEW
_DTAIL_BASE = _N_EDGES + _TAIL_OFF - _DST_SKEW
_DTAIL_LEN = _TAIL + _DST_SKEW

_SC_BASE = _N_EDGES + 8 - _DST_SKEW
_STRAG_TAIL = _N_EDGES * 2 - 60

_mesh = plsc.VectorSubcoreMesh(core_axis_name="c", subcore_axis_name="s")
_params = pltpu.CompilerParams(needs_layout_passes=False)


def _rsqrt16(d):
    ii = plsc.bitcast(d, jnp.int32)
    ii = jnp.int32(0x5F3759DF) - (ii >> 1)
    y = plsc.bitcast(ii, jnp.float32)
    half_d = d * jnp.float32(0.5)
    for _ in range(3):
        y = y * (jnp.float32(1.5) - half_d * y * y)
    return jnp.where(d > jnp.float32(0.5), y, jnp.float32(0.0))


@functools.partial(
    pl.kernel,
    mesh=_mesh,
    out_type=jax.ShapeDtypeStruct((_N_EDGES,), jnp.float32),
    compiler_params=_params,
    scratch_types=[
        pltpu.VMEM((_RPT, 128), jnp.int32),
        pltpu.VMEM((_RPT * 128,), jnp.int32),
        pltpu.VMEM((64,), jnp.int32),
        pltpu.VMEM((80,), jnp.int32),
        pltpu.VMEM((128,), jnp.float32),
        pltpu.VMEM((64,), jnp.float32),
        pltpu.VMEM((_NODES_PT,), jnp.float32),
        pltpu.VMEM((_N_PAD,), jnp.float32),
        pltpu.VMEM((_E_PW,), jnp.int32),
        pltpu.VMEM((_E_PW + 8,), jnp.int32),
        pltpu.VMEM((_E_PW,), jnp.float32),
        pltpu.VMEM((64,), jnp.int32),
        pltpu.VMEM((64,), jnp.float32),
        pltpu.VMEM_SHARED((_N_PAD,), jnp.float32),
        pltpu.SemaphoreType.DMA,
        pltpu.SemaphoreType.DMA,
    ],
)
def _gcn_kernel(flat_hbm, out_hbm,
                dstb_v, dstf_v, trow_v, tdst_v, ones_v, ones64_v, pa_v, w_v,
                src_v, dst_v, out_v, tsrc_v, tout_v, hist_s, sem, esem):
    cid = lax.axis_index("c")
    sid = lax.axis_index("s")
    wid = sid * _NC + cid
    col = sid * _NODES_PT

    scope = jax.named_scope
    @pl.when(sid < _NS - 1)
    def _():
        pltpu.sync_copy(
            flat_hbm.at[pl.ds(_SC_BASE + sid * _RPT * 128, _RPT * 128)],
            dstf_v)

    @pl.when(sid == _NS - 1)
    def _():
        pltpu.sync_copy(
            flat_hbm.at[pl.ds(_SC_BASE + (_NS - 1) * _RPT * 128,
                              _RLAST * 128)],
            dstf_v.at[pl.ds(0, _RLAST * 128)])

    zero16 = jnp.zeros((16,), jnp.float32)
    one16 = jnp.ones((16,), jnp.float32)
    for i in range(_NODES_PT // 16):
        pa_v[pl.ds(i * 16, 16)] = zero16
    for i in range(8):
        ones_v[pl.ds(i * 16, 16)] = one16
    for i in range(4):
        ones64_v[pl.ds(i * 16, 16)] = one16
    sent16 = jnp.full((16,), _SENT, jnp.int32)
    lane16 = lax.iota(jnp.int32, 16)

    @pl.when(sid == 8)
    def _():
        pltpu.sync_copy(flat_hbm.at[pl.ds(_DST_BASE, 8)],
                        tdst_v.at[pl.ds(0, 8)])
        pltpu.sync_copy(flat_hbm.at[pl.ds(_STRAG_TAIL, 60)],
                        tdst_v.at[pl.ds(8, 60)])
        for i in range(4):
            v = tdst_v[pl.ds(_DST_SKEW + i * 16, 16)]
            nvalid = 62 - i * 16
            if nvalid < 16:
                v = jnp.where(lane16 < nvalid, v, sent16)
            trow_v[pl.ds(i * 16, 16)] = v

    base = wid * _E_PW
    pltpu.async_copy(flat_hbm.at[pl.ds(base, _E_PW)], src_v, esem)
    pltpu.async_copy(flat_hbm.at[pl.ds(_DST_BASE + base, _E_PW + 8)],
                     dst_v, esem)

    with scope("sc_stage"):
        pltpu.sync_copy(pa_v, hist_s.at[pl.ds(col, _NODES_PT)])
        plsc.subcore_barrier()

    k = 8
    nch = jnp.where(sid == _NS - 1, _RLAST // k, _RPT // k)

    def _repack(j):
        for r in range(k):
            row = j * k + r
            for u in range(8):
                dstb_v[row, pl.ds(u * 16, 16)] = (
                    dstf_v[pl.ds(row * 128 + u * 16, 16)])

    def _fire(j):
        for u in range(k):
            pltpu.async_copy(ones_v, hist_s.at[dstb_v.at[j * k + u]], sem,
                             add=True)

    def _drain(j):
        for u in range(k):
            pltpu.make_async_copy(ones_v, hist_s.at[dstb_v.at[j * k + u]],
                                  sem).wait()

    with scope("sc_scatter"):
        _repack(0)
        _fire(0)
        _repack(1)
        _fire(1)
        _repack(2)
        _fire(2)

        def scatter_body(j, _):
            _repack(j)
            _fire(j)
            _drain(j - 3)
            return 0

        lax.fori_loop(3, nch, scatter_body, 0)
        _drain(nch - 3)
        _drain(nch - 2)
        _drain(nch - 1)

        @pl.when(sid == 8)
        def _():
            pltpu.sync_copy(ones64_v, hist_s.at[trow_v], add=True)

        plsc.subcore_barrier()

    with scope("sc_wphase"):
        pltpu.sync_copy(hist_s.at[pl.ds(col, _NODES_PT)], pa_v)
        for i in range(_NODES_PT // 16):
            ds = pl.ds(i * 16, 16)
            pa_v[ds] = _rsqrt16(pa_v[ds])
        pltpu.sync_copy(pa_v, hist_s.at[pl.ds(col, _NODES_PT)])
        plsc.subcore_barrier()

    with scope("sc_wcopy"):
        pltpu.sync_copy(hist_s, w_v)
        pltpu.make_async_copy(flat_hbm.at[pl.ds(base, _E_PW)], src_v,
                              esem).wait()
        pltpu.make_async_copy(flat_hbm.at[pl.ds(_DST_BASE + base, _E_PW + 8)],
                              dst_v, esem).wait()

    with scope("sc_gather"):
        @plsc.parallel_loop(0, _E_PW // 16, unroll=16)
        def _(i):
            off = i * 16
            ws = plsc.load_gather(w_v, [src_v[pl.ds(off, 16)]])
            wd = plsc.load_gather(w_v, [dst_v[pl.ds(off + _DST_SKEW, 16)]])
            out_v[pl.ds(off, 16)] = ws * wd

        pltpu.sync_copy(out_v, out_hbm.at[pl.ds(base, _E_PW)])

    @pl.when(wid == _NW - 1)
    def _():
        for i in range(4):
            ds = pl.ds(i * 16, 16)
            tsrc_v[ds] = sent16
        pltpu.sync_copy(flat_hbm.at[pl.ds(_TAIL_OFF, _TAIL)],
                        tsrc_v.at[pl.ds(0, _TAIL)])
        pltpu.sync_copy(flat_hbm.at[pl.ds(_DTAIL_BASE, _DTAIL_LEN)],
                        tdst_v.at[pl.ds(0, _DTAIL_LEN)])
        for i in range(4):
            ds = pl.ds(i * 16, 16)
            sv = tsrc_v[ds]
            dv = tdst_v[pl.ds(_DST_SKEW + i * 16, 16)]
            nvalid = _TAIL - i * 16
            if nvalid < 16:
                dv = jnp.where(lane16 < nvalid, dv, sent16)
            ws = plsc.load_gather(w_v, [sv])
            wd = plsc.load_gather(w_v, [dv])
            tout_v[ds] = ws * wd
        pltpu.sync_copy(tout_v.at[pl.ds(0, _TAIL)],
                        out_hbm.at[pl.ds(_TAIL_OFF, _TAIL)])


def kernel(arg0_1):
    flat = arg0_1.astype(jnp.int32).reshape(-1)
    out = _gcn_kernel(flat)
    return (out,)


# speedup vs baseline: 1.0283x; 1.0283x over previous
import functools

import jax
import jax.numpy as jnp
from jax import lax
from jax.experimental import pallas as pl
from jax.experimental.pallas import tpu as pltpu
from jax.experimental.pallas import tpu_sc as plsc

_N_EDGES = 209982
_N_NODES = 10000
_N_PAD = 10240
_NC = 2
_NS = 16
_NW = _NC * _NS
_SENT = _N_PAD - 1

_NROWS = _N_EDGES // 128
_RPT = 104
_RLAST = _NROWS - _RPT * (_NS - 1)
_TAIL_OFF = _NROWS * 128
_TAIL = _N_EDGES - _TAIL_OFF

_E_PW = 6560
_NODES_PT = _N_PAD // _NS

_DST_SKEW = _N_EDGES % 8
_DST_BASE = _N_EDGES - _DST_SKEW
_DTAIL_BASE = _N_EDGES + _TAIL_OFF - _DST_SKEW
_DTAIL_LEN = _TAIL + _DST_SKEW

_SC_BASE = _N_EDGES + 8 - _DST_SKEW
_STRAG_TAIL = _N_EDGES * 2 - 60

_mesh = plsc.VectorSubcoreMesh(core_axis_name="c", subcore_axis_name="s")
_params = pltpu.CompilerParams(needs_layout_passes=False)


def _rsqrt16(d):
    ii = plsc.bitcast(d, jnp.int32)
    ii = jnp.int32(0x5F3759DF) - (ii >> 1)
    y = plsc.bitcast(ii, jnp.float32)
    half_d = d * jnp.float32(0.5)
    for _ in range(3):
        y = y * (jnp.float32(1.5) - half_d * y * y)
    return jnp.where(d > jnp.float32(0.5), y, jnp.float32(0.0))


@functools.partial(
    pl.kernel,
    mesh=_mesh,
    out_type=jax.ShapeDtypeStruct((_N_EDGES,), jnp.float32),
    compiler_params=_params,
    scratch_types=[
        pltpu.VMEM((_RPT, 128), jnp.int32),
        pltpu.VMEM((_RPT * 128,), jnp.int32),
        pltpu.VMEM((64,), jnp.int32),
        pltpu.VMEM((80,), jnp.int32),
        pltpu.VMEM((128,), jnp.float32),
        pltpu.VMEM((64,), jnp.float32),
        pltpu.VMEM((_NODES_PT,), jnp.float32),
        pltpu.VMEM((_N_PAD,), jnp.float32),
        pltpu.VMEM((_E_PW,), jnp.int32),
        pltpu.VMEM((_E_PW + 8,), jnp.int32),
        pltpu.VMEM((_E_PW,), jnp.float32),
        pltpu.VMEM((64,), jnp.int32),
        pltpu.VMEM((64,), jnp.float32),
        pltpu.VMEM_SHARED((_N_PAD,), jnp.float32),
        pltpu.SemaphoreType.DMA,
        pltpu.SemaphoreType.DMA,
        pltpu.SemaphoreType.DMA,
    ],
)
def _gcn_kernel(flat_hbm, out_hbm,
                dstb_v, dstf_v, trow_v, tdst_v, ones_v, ones64_v, pa_v, w_v,
                src_v, dst_v, out_v, tsrc_v, tout_v, hist_s, sem, esem,
                ssem):
    cid = lax.axis_index("c")
    sid = lax.axis_index("s")
    wid = sid * _NC + cid
    col = sid * _NODES_PT

    scope = jax.named_scope
    @pl.when(sid < _NS - 1)
    def _():
        pltpu.async_copy(
            flat_hbm.at[pl.ds(_SC_BASE + sid * _RPT * 128, _RPT * 128)],
            dstf_v, ssem)

    @pl.when(sid == _NS - 1)
    def _():
        pltpu.async_copy(
            flat_hbm.at[pl.ds(_SC_BASE + (_NS - 1) * _RPT * 128,
                              _RLAST * 128)],
            dstf_v.at[pl.ds(0, _RLAST * 128)], ssem)

    zero16 = jnp.zeros((16,), jnp.float32)
    one16 = jnp.ones((16,), jnp.float32)
    for i in range(_NODES_PT // 16):
        pa_v[pl.ds(i * 16, 16)] = zero16
    for i in range(8):
        ones_v[pl.ds(i * 16, 16)] = one16
    for i in range(4):
        ones64_v[pl.ds(i * 16, 16)] = one16
    sent16 = jnp.full((16,), _SENT, jnp.int32)
    lane16 = lax.iota(jnp.int32, 16)

    @pl.when(sid == 8)
    def _():
        pltpu.sync_copy(flat_hbm.at[pl.ds(_DST_BASE, 8)],
                        tdst_v.at[pl.ds(0, 8)])
        pltpu.sync_copy(flat_hbm.at[pl.ds(_STRAG_TAIL, 60)],
                        tdst_v.at[pl.ds(8, 60)])
        for i in range(4):
            v = tdst_v[pl.ds(_DST_SKEW + i * 16, 16)]
            nvalid = 62 - i * 16
            if nvalid < 16:
                v = jnp.where(lane16 < nvalid, v, sent16)
            trow_v[pl.ds(i * 16, 16)] = v

    base = wid * _E_PW
    pltpu.async_copy(flat_hbm.at[pl.ds(base, _E_PW)], src_v, esem)
    pltpu.async_copy(flat_hbm.at[pl.ds(_DST_BASE + base, _E_PW + 8)],
                     dst_v, esem)

    with scope("sc_stage"):
        pltpu.sync_copy(pa_v, hist_s.at[pl.ds(col, _NODES_PT)])
        plsc.subcore_barrier()

    k = 8
    nch = jnp.where(sid == _NS - 1, _RLAST // k, _RPT // k)

    def _repack(j):
        for r in range(k):
            row = j * k + r
            for u in range(8):
                dstb_v[row, pl.ds(u * 16, 16)] = (
                    dstf_v[pl.ds(row * 128 + u * 16, 16)])

    def _fire(j):
        for u in range(k):
            pltpu.async_copy(ones_v, hist_s.at[dstb_v.at[j * k + u]], sem,
                             add=True)

    def _drain(j):
        for u in range(k):
            pltpu.make_async_copy(ones_v, hist_s.at[dstb_v.at[j * k + u]],
                                  sem).wait()

    with scope("sc_scatter"):
        @pl.when(sid < _NS - 1)
        def _():
            pltpu.make_async_copy(
                flat_hbm.at[pl.ds(_SC_BASE + sid * _RPT * 128, _RPT * 128)],
                dstf_v, ssem).wait()

        @pl.when(sid == _NS - 1)
        def _():
            pltpu.make_async_copy(
                flat_hbm.at[pl.ds(_SC_BASE + (_NS - 1) * _RPT * 128,
                                  _RLAST * 128)],
                dstf_v.at[pl.ds(0, _RLAST * 128)], ssem).wait()

        _repack(0)
        _fire(0)
        _repack(1)
        _fire(1)

        def scatter_body(j, _):
            _repack(j)
            _fire(j)
            _drain(j - 2)
            return 0

        lax.fori_loop(2, nch, scatter_body, 0)
        _drain(nch - 2)
        _drain(nch - 1)

        @pl.when(sid == 8)
        def _():
            pltpu.sync_copy(ones64_v, hist_s.at[trow_v], add=True)

        plsc.subcore_barrier()

    with scope("sc_wphase"):
        pltpu.sync_copy(hist_s.at[pl.ds(col, _NODES_PT)], pa_v)

        @plsc.parallel_loop(0, _NODES_PT // 16, unroll=8)
        def _(i):
            ds = pl.ds(i * 16, 16)
            pa_v[ds] = _rsqrt16(pa_v[ds])

        pltpu.sync_copy(pa_v, hist_s.at[pl.ds(col, _NODES_PT)])
        plsc.subcore_barrier()

    with scope("sc_wcopy"):
        pltpu.sync_copy(hist_s, w_v)
        pltpu.make_async_copy(flat_hbm.at[pl.ds(base, _E_PW)], src_v,
                              esem).wait()
        pltpu.make_async_copy(flat_hbm.at[pl.ds(_DST_BASE + base, _E_PW + 8)],
                              dst_v, esem).wait()

    with scope("sc_gather"):
        @plsc.parallel_loop(0, _E_PW // 16, unroll=8)
        def _(i):
            off = i * 16
            ws = plsc.load_gather(w_v, [src_v[pl.ds(off, 16)]])
            wd = plsc.load_gather(w_v, [dst_v[pl.ds(off + _DST_SKEW, 16)]])
            out_v[pl.ds(off, 16)] = ws * wd

        pltpu.sync_copy(out_v, out_hbm.at[pl.ds(base, _E_PW)])

    @pl.when(wid == _NW - 1)
    def _():
        for i in range(4):
            ds = pl.ds(i * 16, 16)
            tsrc_v[ds] = sent16
        pltpu.sync_copy(flat_hbm.at[pl.ds(_TAIL_OFF, _TAIL)],
                        tsrc_v.at[pl.ds(0, _TAIL)])
        pltpu.sync_copy(flat_hbm.at[pl.ds(_DTAIL_BASE, _DTAIL_LEN)],
                        tdst_v.at[pl.ds(0, _DTAIL_LEN)])
        for i in range(4):
            ds = pl.ds(i * 16, 16)
            sv = tsrc_v[ds]
            dv = tdst_v[pl.ds(_DST_SKEW + i * 16, 16)]
            nvalid = _TAIL - i * 16
            if nvalid < 16:
                dv = jnp.where(lane16 < nvalid, dv, sent16)
            ws = plsc.load_gather(w_v, [sv])
            wd = plsc.load_gather(w_v, [dv])
            tout_v[ds] = ws * wd
        pltpu.sync_copy(tout_v.at[pl.ds(0, _TAIL)],
                        out_hbm.at[pl.ds(_TAIL_OFF, _TAIL)])


def kernel(arg0_1):
    flat = arg0_1.astype(jnp.int32).reshape(-1)
    out = _gcn_kernel(flat)
    return (out,)
